# baseline (speedup 1.0000x reference)
import functools

import jax
import jax.numpy as jnp
from jax import lax
from jax.experimental import pallas as pl
from jax.experimental.pallas import tpu as pltpu
from jax.experimental.pallas import tpu_sc as plsc

N = 10000
NP = 10240
E = 320000
D = 128
G = 64

NC = 2
NS = 16
CH = 40
EPT = E // (NC * NS)
NCHUNK = EPT // CH
RPT = NP // NS
BLK = 512
NBLK = NP // BLK


NBUF = 5
CW = D // NC
EPT2 = E // NS
NCHUNK2 = EPT2 // CH
NGRP = NCHUNK2 // NBUF


def _spmm_body(y_hbm, src_hbm, dst_hbm, out_hbm, acc, stage, *bufs):
    srcs = bufs[0:NBUF]
    dsts = bufs[NBUF:2 * NBUF]
    rows = bufs[2 * NBUF:3 * NBUF]
    gsem = bufs[3 * NBUF:4 * NBUF]
    ssem = bufs[4 * NBUF:5 * NBUF]
    c = lax.axis_index("c")
    s = lax.axis_index("s")
    base_r = s * RPT
    e0 = s * EPT2

    def load_and_gather(b, chunk):
        off = pl.multiple_of(e0 + chunk * CH, 8)
        pltpu.sync_copy(src_hbm.at[pl.ds(off, CH)], srcs[b])
        pltpu.sync_copy(dst_hbm.at[pl.ds(off, CH)], dsts[b])
        pltpu.async_copy(stage.at[srcs[b]], rows[b], gsem[b])

    pltpu.async_copy(y_hbm.at[c, pl.ds(base_r, RPT)],
                     stage.at[pl.ds(base_r, RPT)], gsem[0])

    def zrow(r, carry):
        for j in range(CW // 16):
            rows[0][r, pl.ds(j * 16, 16)] = jnp.zeros((16,), jnp.float32)
        return carry

    lax.fori_loop(0, CH, zrow, 0)

    def zacc(k, carry):
        pltpu.async_copy(rows[0], acc.at[pl.ds(base_r + k * CH, CH)], ssem[0])
        return carry

    lax.fori_loop(0, RPT // CH, zacc, 0)

    def zwait(k, carry):
        pltpu.make_async_copy(rows[0], acc.at[pl.ds(base_r + k * CH, CH)],
                              ssem[0]).wait()
        return carry

    lax.fori_loop(0, RPT // CH, zwait, 0)
    pltpu.make_async_copy(y_hbm.at[c, pl.ds(base_r, RPT)],
                          stage.at[pl.ds(base_r, RPT)], gsem[0]).wait()
    plsc.subcore_barrier()
    for b in range(NBUF):
        load_and_gather(b, b)

    def group(t, carry):
        for b in range(NBUF):
            pltpu.make_async_copy(stage.at[srcs[b]], rows[b], gsem[b]).wait()
            pltpu.async_copy(rows[b], acc.at[dsts[b]], ssem[b], add=True)
        for b in range(NBUF):
            pltpu.make_async_copy(rows[b], acc.at[dsts[b]], ssem[b]).wait()

            @pl.when(t < NGRP - 1)
            def _prefetch():
                load_and_gather(b, NBUF * (t + 1) + b)

        return carry

    lax.fori_loop(0, NGRP, group, 0)
    plsc.subcore_barrier()
    pltpu.sync_copy(acc.at[pl.ds(base_r, RPT)],
                    out_hbm.at[c, pl.ds(base_r, RPT)])


_spmm = functools.partial(
    pl.kernel,
    mesh=plsc.VectorSubcoreMesh(core_axis_name="c", subcore_axis_name="s"),
    out_type=jax.ShapeDtypeStruct((NC, NP, CW), jnp.float32),
    scratch_types=(
        [pltpu.VMEM_SHARED((NP, CW), jnp.float32),
         pltpu.VMEM_SHARED((NP, CW), jnp.float32)]
        + [pltpu.VMEM((CH,), jnp.int32)] * (2 * NBUF)
        + [pltpu.VMEM((CH, CW), jnp.float32)] * NBUF
        + [pltpu.SemaphoreType.DMA] * (2 * NBUF)
    ),
)(_spmm_body)


def _layer_body(relu, p_ref, h_ref, wr_ref, wo_ref, b_ref, o_ref):
    agg = jnp.concatenate([p_ref[0], p_ref[1]], axis=1)
    h = jnp.concatenate([h_ref[0], h_ref[1]], axis=1)
    z = (jnp.dot(agg, wr_ref[...], preferred_element_type=jnp.float32)
         + jnp.dot(h, wo_ref[...], preferred_element_type=jnp.float32)
         + b_ref[...])
    z = jnp.maximum(z, 0.0) if relu else z
    o_ref[0] = z[:, :CW]
    o_ref[1] = z[:, CW:]


_row_spec = pl.BlockSpec((BLK, D), lambda i: (i, 0))
_split_spec = pl.BlockSpec((NC, BLK, CW), lambda i: (0, i, 0))
_w_spec = pl.BlockSpec((D, D), lambda i: (0, 0))
_b_spec = pl.BlockSpec((1, D), lambda i: (0, 0))


def _layer(p, h, wr, wo, b, relu):
    return pl.pallas_call(
        functools.partial(_layer_body, relu),
        grid=(NBLK,),
        in_specs=[_split_spec, _split_spec, _w_spec, _w_spec, _b_spec],
        out_specs=_split_spec,
        out_shape=jax.ShapeDtypeStruct((NC, NP, CW), jnp.float32),
    )(p, h, wr, wo, b)


def _head_body(p_ref, hp_ref, w3r_ref, w3o_ref, b3l_ref,
               bat_ref, t_ref, w1a_ref, w1b_ref,
               w1c_ref, w1d_ref, b1_ref, w2_ref, b2_ref, w3_ref, b3_ref,
               out_ref, max_acc, sum_acc, cnt_acc):
    i = pl.program_id(0)

    @pl.when(i == 0)
    def _init():
        max_acc[...] = jnp.full((G, D), -jnp.inf, jnp.float32)
        sum_acc[...] = jnp.zeros((G, D), jnp.float32)
        cnt_acc[...] = jnp.zeros((G, D), jnp.float32)

    agg = jnp.concatenate([p_ref[0], p_ref[1]], axis=1)
    hp = jnp.concatenate([hp_ref[0], hp_ref[1]], axis=1)
    h = (jnp.dot(agg, w3r_ref[...], preferred_element_type=jnp.float32)
         + jnp.dot(hp, w3o_ref[...], preferred_element_type=jnp.float32)
         + b3l_ref[...])
    bat = bat_ref[...]
    gids = lax.broadcasted_iota(jnp.int32, (BLK, G), 1)
    mask = (bat == gids).astype(jnp.float32)
    dn = (((0,), (0,)), ((), ()))
    sum_acc[...] = sum_acc[...] + lax.dot_general(
        mask, h, dn, preferred_element_type=jnp.float32,
        precision=lax.Precision.HIGHEST)
    cnt_acc[...] = cnt_acc[...] + lax.dot_general(
        mask, jnp.ones((BLK, D), jnp.float32), dn,
        preferred_element_type=jnp.float32, precision=lax.Precision.HIGHEST)
    parts = []
    for g in range(G):
        sel = jnp.where(bat == g, h, -jnp.inf)
        parts.append(jnp.max(sel, axis=0, keepdims=True))
    max_acc[...] = jnp.maximum(max_acc[...], jnp.concatenate(parts, axis=0))

    @pl.when(i == pl.num_programs(0) - 1)
    def _finish():
        maxp = max_acc[...]
        sump = sum_acc[...]
        meanp = sump / jnp.maximum(cnt_acc[...], 1.0)
        z = (jnp.dot(maxp, w1a_ref[...], preferred_element_type=jnp.float32)
             + jnp.dot(meanp, w1b_ref[...], preferred_element_type=jnp.float32)
             + jnp.dot(sump, w1c_ref[...], preferred_element_type=jnp.float32)
             + t_ref[...] * w1d_ref[...] + b1_ref[...])
        z = jnp.maximum(z, 0.0)
        z = jnp.maximum(
            jnp.dot(z, w2_ref[...], preferred_element_type=jnp.float32)
            + b2_ref[...], 0.0)
        out_ref[...] = (jnp.dot(z, w3_ref[...], preferred_element_type=jnp.float32)
                        + b3_ref[...])


def _head(p, hp, w3r, w3o, b3l, batp, T, w1a, w1b, w1c, w1d, b1, w2, b2, w3, b3):
    full = lambda shape: pl.BlockSpec(shape, lambda i: tuple(0 for _ in shape))
    return pl.pallas_call(
        _head_body,
        grid=(NBLK,),
        in_specs=[_split_spec, _split_spec,
                  full((D, D)), full((D, D)), full((1, D)),
                  pl.BlockSpec((BLK, 1), lambda i: (i, 0)),
                  full((G, 1)), full((D, D)), full((D, D)), full((D, D)),
                  full((1, D)), full((1, D)), full((D, D)), full((1, D)),
                  full((D, 1)), full((1, 1))],
        out_specs=full((G, 1)),
        out_shape=jax.ShapeDtypeStruct((G, 1), jnp.float32),
        scratch_shapes=[pltpu.VMEM((G, D), jnp.float32)] * 3,
    )(p, hp, w3r, w3o, b3l, batp, T, w1a, w1b, w1c, w1d, b1, w2, b2, w3, b3)


def kernel(x, edge_index, batch, T, W1_rel, W1_root, b1, W2_rel, W2_root, b2,
           W3_rel, W3_root, b3, lin1_W, lin1_b, lin2_W, lin2_b, lin3_W, lin3_b):
    xp = jnp.pad(x, ((0, NP - N), (0, 0)))
    xs = jnp.stack([xp[:, :CW], xp[:, CW:]])
    src = edge_index[0]
    dst = edge_index[1]
    batp = jnp.pad(batch, (0, NP - N), constant_values=G).reshape(NP, 1)

    p = _spmm(xs, src, dst)
    h = _layer(p, xs, W1_rel, W1_root, b1.reshape(1, D), relu=True)
    p = _spmm(h, src, dst)
    h = _layer(p, h, W2_rel, W2_root, b2.reshape(1, D), relu=True)
    p = _spmm(h, src, dst)

    return _head(p, h, W3_rel, W3_root, b3.reshape(1, D), batp, T,
                 lin1_W[0:D], lin1_W[D:2 * D], lin1_W[2 * D:3 * D],
                 lin1_W[3 * D:].reshape(1, D), lin1_b.reshape(1, D),
                 lin2_W, lin2_b.reshape(1, D), lin3_W, lin3_b.reshape(1, 1))


# speedup vs baseline: 2.2332x; 2.2332x over previous
import functools

import jax
import jax.numpy as jnp
from jax import lax
from jax.experimental import pallas as pl
from jax.experimental.pallas import tpu as pltpu
from jax.experimental.pallas import tpu_sc as plsc

N = 10000
NP = 10240
E = 320000
D = 128
G = 64

NC = 2
NS = 16
CH = 40
EPT = E // (NC * NS)
NCHUNK = EPT // CH
RPT = NP // NS
BLK = 512
NBLK = NP // BLK


NBUF = 5
NGRP = NCHUNK // NBUF


def _spmm_body(y_hbm, src_hbm, dst_hbm, out_hbm, acc, src_all, *bufs):
    dsts = bufs[0:NBUF]
    rows = bufs[NBUF:2 * NBUF]
    gsem = bufs[2 * NBUF:3 * NBUF]
    ssem = bufs[3 * NBUF:4 * NBUF]
    c = lax.axis_index("c")
    s = lax.axis_index("s")
    wid = c * NS + s
    e0 = c * (E // NC) + s * EPT

    pltpu.sync_copy(src_hbm.at[wid], src_all)

    def gather(b, chunk):
        off = pl.multiple_of(chunk * CH, 8)
        pltpu.sync_copy(dst_hbm.at[pl.ds(e0 + off, CH)], dsts[b])
        pltpu.async_copy(y_hbm.at[src_all.at[pl.ds(off, CH)]], rows[b],
                         gsem[b])

    for b in range(1, NBUF):
        gather(b, b)

    def zrow(r, carry):
        for j in range(8):
            rows[0][r, pl.ds(j * 16, 16)] = jnp.zeros((16,), jnp.float32)
        return carry

    lax.fori_loop(0, CH, zrow, 0)
    base_r = s * RPT

    def zacc(k, carry):
        pltpu.async_copy(rows[0], acc.at[pl.ds(base_r + k * CH, CH)], ssem[0])
        return carry

    lax.fori_loop(0, RPT // CH, zacc, 0)

    def zwait(k, carry):
        pltpu.make_async_copy(rows[0], acc.at[pl.ds(base_r + k * CH, CH)],
                              ssem[0]).wait()
        return carry

    lax.fori_loop(0, RPT // CH, zwait, 0)
    gather(0, 0)
    plsc.subcore_barrier()

    def group(t, carry):
        for b in range(NBUF):
            chunk = NBUF * t + b
            off = pl.multiple_of(chunk * CH, 8)
            pltpu.make_async_copy(y_hbm.at[src_all.at[pl.ds(off, CH)]],
                                  rows[b], gsem[b]).wait()
            pltpu.async_copy(rows[b], acc.at[dsts[b]], ssem[b], add=True)
        for b in range(NBUF):
            pltpu.make_async_copy(rows[b], acc.at[dsts[b]], ssem[b]).wait()

            @pl.when(t < NGRP - 1)
            def _prefetch():
                gather(b, NBUF * (t + 1) + b)

        return carry

    lax.fori_loop(0, NGRP, group, 0)
    plsc.subcore_barrier()
    pltpu.sync_copy(acc.at[pl.ds(base_r, RPT)], out_hbm.at[c, pl.ds(base_r, RPT)])


_spmm = functools.partial(
    pl.kernel,
    mesh=plsc.VectorSubcoreMesh(core_axis_name="c", subcore_axis_name="s"),
    out_type=jax.ShapeDtypeStruct((NC, NP, D), jnp.float32),
    scratch_types=(
        [pltpu.VMEM_SHARED((NP, D), jnp.float32),
         pltpu.VMEM((EPT,), jnp.int32)]
        + [pltpu.VMEM((CH,), jnp.int32)] * NBUF
        + [pltpu.VMEM((CH, D), jnp.float32)] * NBUF
        + [pltpu.SemaphoreType.DMA] * (2 * NBUF)
    ),
)(_spmm_body)


def _layer_body(relu, p0_ref, p1_ref, h_ref, wr_ref, wo_ref, b_ref, o_ref):
    agg = p0_ref[...] + p1_ref[...]
    z = (jnp.dot(agg, wr_ref[...], preferred_element_type=jnp.float32)
         + jnp.dot(h_ref[...], wo_ref[...], preferred_element_type=jnp.float32)
         + b_ref[...])
    o_ref[...] = jnp.maximum(z, 0.0) if relu else z


_row_spec = pl.BlockSpec((BLK, D), lambda i: (i, 0))
_w_spec = pl.BlockSpec((D, D), lambda i: (0, 0))
_b_spec = pl.BlockSpec((1, D), lambda i: (0, 0))


def _layer(pp, h, wr, wo, b, relu):
    return pl.pallas_call(
        functools.partial(_layer_body, relu),
        grid=(NBLK,),
        in_specs=[_row_spec, _row_spec, _row_spec, _w_spec, _w_spec, _b_spec],
        out_specs=_row_spec,
        out_shape=jax.ShapeDtypeStruct((NP, D), jnp.float32),
    )(pp[0], pp[1], h, wr, wo, b)


def _head_body(p0_ref, p1_ref, hp_ref, w3r_ref, w3o_ref, b3l_ref,
               bat_ref, t_ref, w1a_ref, w1b_ref,
               w1c_ref, w1d_ref, b1_ref, w2_ref, b2_ref, w3_ref, b3_ref,
               out_ref, max_acc, sum_acc, cnt_acc):
    i = pl.program_id(0)

    @pl.when(i == 0)
    def _init():
        max_acc[...] = jnp.full((G, D), -jnp.inf, jnp.float32)
        sum_acc[...] = jnp.zeros((G, D), jnp.float32)
        cnt_acc[...] = jnp.zeros((G, D), jnp.float32)

    agg = p0_ref[...] + p1_ref[...]
    h = (jnp.dot(agg, w3r_ref[...], preferred_element_type=jnp.float32)
         + jnp.dot(hp_ref[...], w3o_ref[...], preferred_element_type=jnp.float32)
         + b3l_ref[...])
    bat = bat_ref[...]
    gids = lax.broadcasted_iota(jnp.int32, (BLK, G), 1)
    mask = (bat == gids).astype(jnp.float32)
    dn = (((0,), (0,)), ((), ()))
    sum_acc[...] = sum_acc[...] + lax.dot_general(
        mask, h, dn, preferred_element_type=jnp.float32,
        precision=lax.Precision.HIGHEST)
    cnt_acc[...] = cnt_acc[...] + lax.dot_general(
        mask, jnp.ones((BLK, D), jnp.float32), dn,
        preferred_element_type=jnp.float32, precision=lax.Precision.HIGHEST)
    parts = []
    for g in range(G):
        sel = jnp.where(bat == g, h, -jnp.inf)
        parts.append(jnp.max(sel, axis=0, keepdims=True))
    max_acc[...] = jnp.maximum(max_acc[...], jnp.concatenate(parts, axis=0))

    @pl.when(i == pl.num_programs(0) - 1)
    def _finish():
        maxp = max_acc[...]
        sump = sum_acc[...]
        meanp = sump / jnp.maximum(cnt_acc[...], 1.0)
        z = (jnp.dot(maxp, w1a_ref[...], preferred_element_type=jnp.float32)
             + jnp.dot(meanp, w1b_ref[...], preferred_element_type=jnp.float32)
             + jnp.dot(sump, w1c_ref[...], preferred_element_type=jnp.float32)
             + t_ref[...] * w1d_ref[...] + b1_ref[...])
        z = jnp.maximum(z, 0.0)
        z = jnp.maximum(
            jnp.dot(z, w2_ref[...], preferred_element_type=jnp.float32)
            + b2_ref[...], 0.0)
        out_ref[...] = (jnp.dot(z, w3_ref[...], preferred_element_type=jnp.float32)
                        + b3_ref[...])


def _head(pp, hp, w3r, w3o, b3l, batp, T, w1a, w1b, w1c, w1d, b1, w2, b2, w3, b3):
    full = lambda shape: pl.BlockSpec(shape, lambda i: tuple(0 for _ in shape))
    return pl.pallas_call(
        _head_body,
        grid=(NBLK,),
        in_specs=[_row_spec, _row_spec, _row_spec,
                  full((D, D)), full((D, D)), full((1, D)),
                  pl.BlockSpec((BLK, 1), lambda i: (i, 0)),
                  full((G, 1)), full((D, D)), full((D, D)), full((D, D)),
                  full((1, D)), full((1, D)), full((D, D)), full((1, D)),
                  full((D, 1)), full((1, 1))],
        out_specs=full((G, 1)),
        out_shape=jax.ShapeDtypeStruct((G, 1), jnp.float32),
        scratch_shapes=[pltpu.VMEM((G, D), jnp.float32)] * 3,
    )(pp[0], pp[1], hp, w3r, w3o, b3l, batp, T, w1a, w1b, w1c, w1d, b1, w2, b2, w3, b3)


def kernel(x, edge_index, batch, T, W1_rel, W1_root, b1, W2_rel, W2_root, b2,
           W3_rel, W3_root, b3, lin1_W, lin1_b, lin2_W, lin2_b, lin3_W, lin3_b):
    xp = jnp.pad(x, ((0, NP - N), (0, 0)))
    src = edge_index[0].reshape(NC * NS, EPT)
    dst = edge_index[1]
    batp = jnp.pad(batch, (0, NP - N), constant_values=G).reshape(NP, 1)

    pp = _spmm(xp, src, dst)
    h = _layer(pp, xp, W1_rel, W1_root, b1.reshape(1, D), relu=True)
    pp = _spmm(h, src, dst)
    h = _layer(pp, h, W2_rel, W2_root, b2.reshape(1, D), relu=True)
    pp = _spmm(h, src, dst)

    return _head(pp, h, W3_rel, W3_root, b3.reshape(1, D), batp, T,
                 lin1_W[0:D], lin1_W[D:2 * D], lin1_W[2 * D:3 * D],
                 lin1_W[3 * D:].reshape(1, D), lin1_b.reshape(1, D),
                 lin2_W, lin2_b.reshape(1, D), lin3_W, lin3_b.reshape(1, 1))


# speedup vs baseline: 2.9628x; 1.3267x over previous
import functools

import jax
import jax.numpy as jnp
from jax import lax
from jax.experimental import pallas as pl
from jax.experimental.pallas import tpu as pltpu
from jax.experimental.pallas import tpu_sc as plsc

N = 10000
NP = 10240
E = 320000
D = 128
G = 64

NC = 2
NS = 16
CH = 40
EPT = E // (NC * NS)
NCHUNK = EPT // CH
RPT = NP // NS
BLK = 512
NBLK = NP // BLK


NBUF = 5
NGRP = NCHUNK // NBUF


def _spmm_body(y_hbm, src_hbm, dst_hbm, out_hbm, acc, src_all, *bufs):
    dsts = bufs[0:NBUF]
    rows = bufs[NBUF:2 * NBUF]
    gsem = bufs[2 * NBUF:3 * NBUF]
    ssem = bufs[3 * NBUF:4 * NBUF]
    dsem = bufs[4 * NBUF:5 * NBUF]
    c = lax.axis_index("c")
    s = lax.axis_index("s")
    wid = c * NS + s
    e0 = c * (E // NC) + s * EPT

    pltpu.sync_copy(src_hbm.at[wid], src_all)

    def gather(b, chunk):
        off = pl.multiple_of(chunk * CH, 8)
        pltpu.async_copy(dst_hbm.at[pl.ds(e0 + off, CH)], dsts[b], dsem[b])
        pltpu.async_copy(y_hbm.at[src_all.at[pl.ds(off, CH)]], rows[b],
                         gsem[b])

    for b in range(1, NBUF):
        gather(b, b)

    def zrow(r, carry):
        for j in range(8):
            rows[0][r, pl.ds(j * 16, 16)] = jnp.zeros((16,), jnp.float32)
        return carry

    lax.fori_loop(0, CH, zrow, 0)
    base_r = s * RPT

    def zacc(k, carry):
        pltpu.async_copy(rows[0], acc.at[pl.ds(base_r + k * CH, CH)], ssem[0])
        return carry

    lax.fori_loop(0, RPT // CH, zacc, 0)

    def zwait(k, carry):
        pltpu.make_async_copy(rows[0], acc.at[pl.ds(base_r + k * CH, CH)],
                              ssem[0]).wait()
        return carry

    lax.fori_loop(0, RPT // CH, zwait, 0)
    gather(0, 0)
    plsc.subcore_barrier()

    def group(t, carry):
        for b in range(NBUF):
            chunk = NBUF * t + b
            off = pl.multiple_of(chunk * CH, 8)
            pltpu.make_async_copy(dst_hbm.at[pl.ds(e0 + off, CH)], dsts[b],
                                  dsem[b]).wait()
            pltpu.make_async_copy(y_hbm.at[src_all.at[pl.ds(off, CH)]],
                                  rows[b], gsem[b]).wait()
            pltpu.async_copy(rows[b], acc.at[dsts[b]], ssem[b], add=True)
        for b in range(NBUF):
            pltpu.make_async_copy(rows[b], acc.at[dsts[b]], ssem[b]).wait()

            @pl.when(t < NGRP - 1)
            def _prefetch():
                gather(b, NBUF * (t + 1) + b)

        return carry

    lax.fori_loop(0, NGRP, group, 0)
    plsc.subcore_barrier()
    pltpu.sync_copy(acc.at[pl.ds(base_r, RPT)], out_hbm.at[c, pl.ds(base_r, RPT)])


_spmm = functools.partial(
    pl.kernel,
    mesh=plsc.VectorSubcoreMesh(core_axis_name="c", subcore_axis_name="s"),
    out_type=jax.ShapeDtypeStruct((NC, NP, D), jnp.float32),
    scratch_types=(
        [pltpu.VMEM_SHARED((NP, D), jnp.float32),
         pltpu.VMEM((EPT,), jnp.int32)]
        + [pltpu.VMEM((CH,), jnp.int32)] * NBUF
        + [pltpu.VMEM((CH, D), jnp.float32)] * NBUF
        + [pltpu.SemaphoreType.DMA] * (3 * NBUF)
    ),
)(_spmm_body)


def _layer_body(relu, p0_ref, p1_ref, h_ref, wr_ref, wo_ref, b_ref, o_ref):
    agg = p0_ref[...] + p1_ref[...]
    z = (jnp.dot(agg, wr_ref[...], preferred_element_type=jnp.float32)
         + jnp.dot(h_ref[...], wo_ref[...], preferred_element_type=jnp.float32)
         + b_ref[...])
    o_ref[...] = jnp.maximum(z, 0.0) if relu else z


_row_spec = pl.BlockSpec((BLK, D), lambda i: (i, 0))
_w_spec = pl.BlockSpec((D, D), lambda i: (0, 0))
_b_spec = pl.BlockSpec((1, D), lambda i: (0, 0))


def _layer(pp, h, wr, wo, b, relu):
    return pl.pallas_call(
        functools.partial(_layer_body, relu),
        grid=(NBLK,),
        in_specs=[_row_spec, _row_spec, _row_spec, _w_spec, _w_spec, _b_spec],
        out_specs=_row_spec,
        out_shape=jax.ShapeDtypeStruct((NP, D), jnp.float32),
    )(pp[0], pp[1], h, wr, wo, b)


def _head_body(p0_ref, p1_ref, hp_ref, w3r_ref, w3o_ref, b3l_ref,
               bat_ref, t_ref, w1a_ref, w1b_ref,
               w1c_ref, w1d_ref, b1_ref, w2_ref, b2_ref, w3_ref, b3_ref,
               out_ref, max_acc, sum_acc, cnt_acc):
    i = pl.program_id(0)

    @pl.when(i == 0)
    def _init():
        max_acc[...] = jnp.full((G, D), -jnp.inf, jnp.float32)
        sum_acc[...] = jnp.zeros((G, D), jnp.float32)
        cnt_acc[...] = jnp.zeros((G, D), jnp.float32)

    agg = p0_ref[...] + p1_ref[...]
    h = (jnp.dot(agg, w3r_ref[...], preferred_element_type=jnp.float32)
         + jnp.dot(hp_ref[...], w3o_ref[...], preferred_element_type=jnp.float32)
         + b3l_ref[...])
    bat = bat_ref[...]
    gids = lax.broadcasted_iota(jnp.int32, (BLK, G), 1)
    mask = (bat == gids).astype(jnp.float32)
    dn = (((0,), (0,)), ((), ()))
    sum_acc[...] = sum_acc[...] + lax.dot_general(
        mask, h, dn, preferred_element_type=jnp.float32,
        precision=lax.Precision.HIGHEST)
    cnt_acc[...] = cnt_acc[...] + lax.dot_general(
        mask, jnp.ones((BLK, D), jnp.float32), dn,
        preferred_element_type=jnp.float32, precision=lax.Precision.HIGHEST)
    parts = []
    for g in range(G):
        sel = jnp.where(bat == g, h, -jnp.inf)
        parts.append(jnp.max(sel, axis=0, keepdims=True))
    max_acc[...] = jnp.maximum(max_acc[...], jnp.concatenate(parts, axis=0))

    @pl.when(i == pl.num_programs(0) - 1)
    def _finish():
        maxp = max_acc[...]
        sump = sum_acc[...]
        meanp = sump / jnp.maximum(cnt_acc[...], 1.0)
        z = (jnp.dot(maxp, w1a_ref[...], preferred_element_type=jnp.float32)
             + jnp.dot(meanp, w1b_ref[...], preferred_element_type=jnp.float32)
             + jnp.dot(sump, w1c_ref[...], preferred_element_type=jnp.float32)
             + t_ref[...] * w1d_ref[...] + b1_ref[...])
        z = jnp.maximum(z, 0.0)
        z = jnp.maximum(
            jnp.dot(z, w2_ref[...], preferred_element_type=jnp.float32)
            + b2_ref[...], 0.0)
        out_ref[...] = (jnp.dot(z, w3_ref[...], preferred_element_type=jnp.float32)
                        + b3_ref[...])


def _head(pp, hp, w3r, w3o, b3l, batp, T, w1a, w1b, w1c, w1d, b1, w2, b2, w3, b3):
    full = lambda shape: pl.BlockSpec(shape, lambda i: tuple(0 for _ in shape))
    return pl.pallas_call(
        _head_body,
        grid=(NBLK,),
        in_specs=[_row_spec, _row_spec, _row_spec,
                  full((D, D)), full((D, D)), full((1, D)),
                  pl.BlockSpec((BLK, 1), lambda i: (i, 0)),
                  full((G, 1)), full((D, D)), full((D, D)), full((D, D)),
                  full((1, D)), full((1, D)), full((D, D)), full((1, D)),
                  full((D, 1)), full((1, 1))],
        out_specs=full((G, 1)),
        out_shape=jax.ShapeDtypeStruct((G, 1), jnp.float32),
        scratch_shapes=[pltpu.VMEM((G, D), jnp.float32)] * 3,
    )(pp[0], pp[1], hp, w3r, w3o, b3l, batp, T, w1a, w1b, w1c, w1d, b1, w2, b2, w3, b3)


def kernel(x, edge_index, batch, T, W1_rel, W1_root, b1, W2_rel, W2_root, b2,
           W3_rel, W3_root, b3, lin1_W, lin1_b, lin2_W, lin2_b, lin3_W, lin3_b):
    xp = jnp.pad(x, ((0, NP - N), (0, 0)))
    src = edge_index[0].reshape(NC * NS, EPT)
    dst = edge_index[1]
    batp = jnp.pad(batch, (0, NP - N), constant_values=G).reshape(NP, 1)

    pp = _spmm(xp, src, dst)
    h = _layer(pp, xp, W1_rel, W1_root, b1.reshape(1, D), relu=True)
    pp = _spmm(h, src, dst)
    h = _layer(pp, h, W2_rel, W2_root, b2.reshape(1, D), relu=True)
    pp = _spmm(h, src, dst)

    return _head(pp, h, W3_rel, W3_root, b3.reshape(1, D), batp, T,
                 lin1_W[0:D], lin1_W[D:2 * D], lin1_W[2 * D:3 * D],
                 lin1_W[3 * D:].reshape(1, D), lin1_b.reshape(1, D),
                 lin2_W, lin2_b.reshape(1, D), lin3_W, lin3_b.reshape(1, 1))


# speedup vs baseline: 3.0066x; 1.0148x over previous
import functools

import jax
import jax.numpy as jnp
from jax import lax
from jax.experimental import pallas as pl
from jax.experimental.pallas import tpu as pltpu
from jax.experimental.pallas import tpu_sc as plsc

N = 10000
NP = 10240
E = 320000
D = 128
G = 64

NC = 2
NS = 16
CH = 40
EPT = E // (NC * NS)
NCHUNK = EPT // CH
RPT = NP // NS
BLK = 512
NBLK = NP // BLK


NBUF = 5
NGRP = NCHUNK // NBUF


def _spmm_body(y_hbm, src_hbm, dst_hbm, out_hbm, acc, src_all, *bufs):
    dsts = bufs[0:NBUF]
    rows = bufs[NBUF:2 * NBUF]
    gsem = bufs[2 * NBUF:3 * NBUF]
    ssem = bufs[3 * NBUF:4 * NBUF]
    dsem = bufs[4 * NBUF:5 * NBUF]
    c = lax.axis_index("c")
    s = lax.axis_index("s")
    wid = c * NS + s
    e0 = c * (E // NC) + s * EPT

    pltpu.sync_copy(src_hbm.at[wid], src_all)

    def gather(b, chunk):
        off = pl.multiple_of(chunk * CH, 8)
        pltpu.async_copy(dst_hbm.at[pl.ds(e0 + off, CH)], dsts[b], dsem[b])
        pltpu.async_copy(y_hbm.at[src_all.at[pl.ds(off, CH)]], rows[b],
                         gsem[b])

    for b in range(1, NBUF):
        gather(b, b)

    def zrow(r, carry):
        for j in range(8):
            rows[0][r, pl.ds(j * 16, 16)] = jnp.zeros((16,), jnp.float32)
        return carry

    lax.fori_loop(0, CH, zrow, 0)
    base_r = s * RPT

    def zacc(k, carry):
        pltpu.async_copy(rows[0], acc.at[pl.ds(base_r + k * CH, CH)], ssem[0])
        return carry

    lax.fori_loop(0, RPT // CH, zacc, 0)

    def zwait(k, carry):
        pltpu.make_async_copy(rows[0], acc.at[pl.ds(base_r + k * CH, CH)],
                              ssem[0]).wait()
        return carry

    lax.fori_loop(0, RPT // CH, zwait, 0)
    gather(0, 0)
    plsc.subcore_barrier()

    def group(t, carry):
        for b in range(NBUF):
            chunk = NBUF * t + b
            off = pl.multiple_of(chunk * CH, 8)
            pltpu.make_async_copy(dst_hbm.at[pl.ds(e0 + off, CH)], dsts[b],
                                  dsem[b]).wait()
            pltpu.make_async_copy(y_hbm.at[src_all.at[pl.ds(off, CH)]],
                                  rows[b], gsem[b]).wait()
            pltpu.async_copy(rows[b], acc.at[dsts[b]], ssem[b], add=True)
        for b in range(NBUF):
            pltpu.make_async_copy(rows[b], acc.at[dsts[b]], ssem[b]).wait()

            @pl.when(t < NGRP - 1)
            def _prefetch():
                gather(b, NBUF * (t + 1) + b)

        return carry

    lax.fori_loop(0, NGRP, group, 0)
    plsc.subcore_barrier()
    pltpu.sync_copy(acc.at[pl.ds(base_r, RPT)], out_hbm.at[c, pl.ds(base_r, RPT)])


_spmm = functools.partial(
    pl.kernel,
    mesh=plsc.VectorSubcoreMesh(core_axis_name="c", subcore_axis_name="s"),
    out_type=jax.ShapeDtypeStruct((NC, NP, D), jnp.float32),
    scratch_types=(
        [pltpu.VMEM_SHARED((NP, D), jnp.float32),
         pltpu.VMEM((EPT,), jnp.int32)]
        + [pltpu.VMEM((CH,), jnp.int32)] * NBUF
        + [pltpu.VMEM((CH, D), jnp.float32)] * NBUF
        + [pltpu.SemaphoreType.DMA] * (3 * NBUF)
    ),
)(_spmm_body)


def _layer_body(relu, p0_ref, p1_ref, h_ref, wr_ref, wo_ref, b_ref, o_ref):
    agg = p0_ref[...] + p1_ref[...]
    z = (jnp.dot(agg, wr_ref[...], preferred_element_type=jnp.float32)
         + jnp.dot(h_ref[...], wo_ref[...], preferred_element_type=jnp.float32)
         + b_ref[...])
    o_ref[...] = jnp.maximum(z, 0.0) if relu else z


_row_spec = pl.BlockSpec((BLK, D), lambda i: (i, 0))
_w_spec = pl.BlockSpec((D, D), lambda i: (0, 0))
_b_spec = pl.BlockSpec((1, D), lambda i: (0, 0))


def _layer(pp, h, wr, wo, b, relu):
    return pl.pallas_call(
        functools.partial(_layer_body, relu),
        grid=(NBLK,),
        in_specs=[_row_spec, _row_spec, _row_spec, _w_spec, _w_spec, _b_spec],
        out_specs=_row_spec,
        out_shape=jax.ShapeDtypeStruct((NP, D), jnp.float32),
    )(pp[0], pp[1], h, wr, wo, b)


def _head_body(p0_ref, p1_ref, hp_ref, w3r_ref, w3o_ref, b3l_ref,
               bat_ref, bats_ref, t_ref, w1a_ref, w1b_ref,
               w1c_ref, w1d_ref, b1_ref, w2_ref, b2_ref, w3_ref, b3_ref,
               out_ref, max_acc, sum_acc, cnt_acc):
    i = pl.program_id(0)

    @pl.when(i == 0)
    def _init():
        max_acc[...] = jnp.full((G, D), -jnp.inf, jnp.float32)
        sum_acc[...] = jnp.zeros((G, D), jnp.float32)
        cnt_acc[...] = jnp.zeros((G, D), jnp.float32)

    agg = p0_ref[...] + p1_ref[...]
    h = (jnp.dot(agg, w3r_ref[...], preferred_element_type=jnp.float32)
         + jnp.dot(hp_ref[...], w3o_ref[...], preferred_element_type=jnp.float32)
         + b3l_ref[...])
    bat = bat_ref[...]
    gids = lax.broadcasted_iota(jnp.int32, (BLK, G), 1)
    mask = (bat == gids).astype(jnp.float32)
    dn = (((0,), (0,)), ((), ()))
    sum_acc[...] = sum_acc[...] + lax.dot_general(
        mask, h, dn, preferred_element_type=jnp.float32,
        precision=lax.Precision.HIGHEST)
    cnt_acc[...] = cnt_acc[...] + lax.dot_general(
        mask, jnp.ones((BLK, D), jnp.float32), dn,
        preferred_element_type=jnp.float32, precision=lax.Precision.HIGHEST)
    lo = bats_ref[0, 0]
    hi = jnp.minimum(bats_ref[BLK - 1, 0], G - 1)

    def gmax(g, carry):
        sel = jnp.where(bat == g, h, -jnp.inf)
        m = jnp.max(sel, axis=0, keepdims=True)
        max_acc[pl.ds(g, 1), :] = jnp.maximum(max_acc[pl.ds(g, 1), :], m)
        return carry

    lax.fori_loop(lo, hi + 1, gmax, 0)

    @pl.when(i == pl.num_programs(0) - 1)
    def _finish():
        maxp = max_acc[...]
        sump = sum_acc[...]
        meanp = sump / jnp.maximum(cnt_acc[...], 1.0)
        z = (jnp.dot(maxp, w1a_ref[...], preferred_element_type=jnp.float32)
             + jnp.dot(meanp, w1b_ref[...], preferred_element_type=jnp.float32)
             + jnp.dot(sump, w1c_ref[...], preferred_element_type=jnp.float32)
             + t_ref[...] * w1d_ref[...] + b1_ref[...])
        z = jnp.maximum(z, 0.0)
        z = jnp.maximum(
            jnp.dot(z, w2_ref[...], preferred_element_type=jnp.float32)
            + b2_ref[...], 0.0)
        out_ref[...] = (jnp.dot(z, w3_ref[...], preferred_element_type=jnp.float32)
                        + b3_ref[...])


def _head(pp, hp, w3r, w3o, b3l, batp, T, w1a, w1b, w1c, w1d, b1, w2, b2, w3, b3):
    full = lambda shape: pl.BlockSpec(shape, lambda i: tuple(0 for _ in shape))
    return pl.pallas_call(
        _head_body,
        grid=(NBLK,),
        in_specs=[_row_spec, _row_spec, _row_spec,
                  full((D, D)), full((D, D)), full((1, D)),
                  pl.BlockSpec((BLK, 1), lambda i: (i, 0)),
                  pl.BlockSpec((BLK, 1), lambda i: (i, 0),
                               memory_space=pltpu.SMEM),
                  full((G, 1)), full((D, D)), full((D, D)), full((D, D)),
                  full((1, D)), full((1, D)), full((D, D)), full((1, D)),
                  full((D, 1)), full((1, 1))],
        out_specs=full((G, 1)),
        out_shape=jax.ShapeDtypeStruct((G, 1), jnp.float32),
        scratch_shapes=[pltpu.VMEM((G, D), jnp.float32)] * 3,
    )(pp[0], pp[1], hp, w3r, w3o, b3l, batp, batp, T, w1a, w1b, w1c, w1d, b1, w2, b2, w3, b3)


def kernel(x, edge_index, batch, T, W1_rel, W1_root, b1, W2_rel, W2_root, b2,
           W3_rel, W3_root, b3, lin1_W, lin1_b, lin2_W, lin2_b, lin3_W, lin3_b):
    xp = jnp.pad(x, ((0, NP - N), (0, 0)))
    src = edge_index[0].reshape(NC * NS, EPT)
    dst = edge_index[1]
    batp = jnp.pad(batch, (0, NP - N), constant_values=G).reshape(NP, 1)

    pp = _spmm(xp, src, dst)
    h = _layer(pp, xp, W1_rel, W1_root, b1.reshape(1, D), relu=True)
    pp = _spmm(h, src, dst)
    h = _layer(pp, h, W2_rel, W2_root, b2.reshape(1, D), relu=True)
    pp = _spmm(h, src, dst)

    return _head(pp, h, W3_rel, W3_root, b3.reshape(1, D), batp, T,
                 lin1_W[0:D], lin1_W[D:2 * D], lin1_W[2 * D:3 * D],
                 lin1_W[3 * D:].reshape(1, D), lin1_b.reshape(1, D),
                 lin2_W, lin2_b.reshape(1, D), lin3_W, lin3_b.reshape(1, 1))
